# Initial kernel scaffold; baseline (speedup 1.0000x reference)
import functools

import jax
import jax.numpy as jnp
from jax import lax
from jax.experimental import pallas as pl
from jax.experimental.pallas import tpu as pltpu
from jax.experimental.pallas import tpu_sc as plsc

EMBED = 32
ROWS = 16384
COLS = 26
B = ROWS * COLS

_info = plsc.get_sparse_core_info()
NC, NS, L = _info.num_cores, _info.num_subcores, _info.num_lanes
NW = NC * NS
B_PER_W = B // NW
CHUNK = 512
N_CHUNKS = B_PER_W // CHUNK


def _body(table_hbm, idx_hbm, out_hbm, idx_v, rows_v, sem):
    wid = lax.axis_index("s") * NC + lax.axis_index("c")
    base = wid * B_PER_W

    def chunk_step(g, _):
        cb = base + g * CHUNK
        pltpu.sync_copy(idx_hbm.at[pl.ds(cb, CHUNK)], idx_v)
        pltpu.async_copy(table_hbm.at[idx_v], rows_v, sem).wait()

        def group_step(i, _):
            v = idx_v[pl.ds(i * L, L)]
            mask = v == 0
            npad = jnp.sum(mask.astype(jnp.int32))

            @pl.when(npad > 0)
            def _fix():
                rvec = i * L + lax.iota(jnp.int32, L)
                zeros = jnp.zeros((L,), jnp.float32)
                for col in range(EMBED):
                    cvec = jnp.full((L,), col, jnp.int32)
                    plsc.store_scatter(rows_v, [rvec, cvec], zeros, mask=mask)

            return 0

        lax.fori_loop(0, CHUNK // L, group_step, 0)
        pltpu.sync_copy(rows_v, out_hbm.at[pl.ds(cb, CHUNK)])
        return 0

    lax.fori_loop(0, N_CHUNKS, chunk_step, 0)


@jax.jit
def _gather(table, idx):
    mesh = plsc.VectorSubcoreMesh(core_axis_name="c", subcore_axis_name="s")
    return pl.kernel(
        _body,
        out_type=jax.ShapeDtypeStruct((B, EMBED), jnp.float32),
        mesh=mesh,
        scratch_types=[
            pltpu.VMEM((CHUNK,), jnp.int32),
            pltpu.VMEM((CHUNK, EMBED), jnp.float32),
            pltpu.SemaphoreType.DMA,
        ],
    )(table, idx)


def kernel(x, table):
    idx = x.reshape(-1).astype(jnp.int32)
    out = _gather(table, idx)
    return out.reshape(ROWS, COLS, EMBED)


# speedup vs baseline: 1.5734x; 1.5734x over previous
import functools

import jax
import jax.numpy as jnp
from jax import lax
from jax.experimental import pallas as pl
from jax.experimental.pallas import tpu as pltpu
from jax.experimental.pallas import tpu_sc as plsc

EMBED = 32
ROWS = 16384
COLS = 26
B = ROWS * COLS

_info = plsc.get_sparse_core_info()
NC, NS, L = _info.num_cores, _info.num_subcores, _info.num_lanes
NW = NC * NS
B_PER_W = B // NW
CHUNK = 512
N_CHUNKS = B_PER_W // CHUNK


def _body(table_hbm, idx_hbm, out_hbm, idx_v, rows_v, sem):
    wid = lax.axis_index("s") * NC + lax.axis_index("c")
    base = wid * B_PER_W

    def chunk_step(g, _):
        cb = base + g * CHUNK
        pltpu.sync_copy(idx_hbm.at[pl.ds(cb, CHUNK)], idx_v)
        pltpu.async_copy(table_hbm.at[idx_v], rows_v, sem).wait()

        def group_step(i, _):
            v = idx_v[pl.ds(i * L, L)]
            m = jnp.where(v == 0, 0.0, 1.0)
            for r in range(L):
                b = m[jnp.full((L,), r, jnp.int32)]
                row = i * L + r
                rows_v[row, pl.ds(0, L)] = rows_v[row, pl.ds(0, L)] * b
                rows_v[row, pl.ds(L, L)] = rows_v[row, pl.ds(L, L)] * b
            return 0

        lax.fori_loop(0, CHUNK // L, group_step, 0)
        pltpu.sync_copy(rows_v, out_hbm.at[pl.ds(cb, CHUNK)])
        return 0

    lax.fori_loop(0, N_CHUNKS, chunk_step, 0)


@jax.jit
def _gather(table, idx):
    mesh = plsc.VectorSubcoreMesh(core_axis_name="c", subcore_axis_name="s")
    return pl.kernel(
        _body,
        out_type=jax.ShapeDtypeStruct((B, EMBED), jnp.float32),
        mesh=mesh,
        scratch_types=[
            pltpu.VMEM((CHUNK,), jnp.int32),
            pltpu.VMEM((CHUNK, EMBED), jnp.float32),
            pltpu.SemaphoreType.DMA,
        ],
        compiler_params=pltpu.CompilerParams(use_tc_tiling_on_sc=False),
    )(table, idx)


def kernel(x, table):
    idx = x.reshape(-1).astype(jnp.int32)
    out = _gather(table, idx)
    return out.reshape(ROWS, COLS, EMBED)


# speedup vs baseline: 1.6202x; 1.0298x over previous
import jax
import jax.numpy as jnp
from jax import lax
from jax.experimental import pallas as pl
from jax.experimental.pallas import tpu as pltpu
from jax.experimental.pallas import tpu_sc as plsc

EMBED = 32
ROWS = 16384
COLS = 26

_info = plsc.get_sparse_core_info()
NC, NS, L = _info.num_cores, _info.num_subcores, _info.num_lanes
NW = NC * NS
ROWS_PER_W = ROWS // NW
R = 64
N_CHUNKS = ROWS_PER_W // R


def _body(table_hbm, x_hbm, out_hbm, idx_v, rows_v, sem):
    wid = lax.axis_index("s") * NC + lax.axis_index("c")
    base = wid * ROWS_PER_W

    def chunk_step(g, _):
        cb = base + g * R
        pltpu.sync_copy(x_hbm.at[pl.ds(cb, R)], idx_v)

        def fire(r, _):
            pltpu.async_copy(table_hbm.at[idx_v.at[r]], rows_v.at[r], sem)
            return 0

        lax.fori_loop(0, R, fire, 0)

        def drain(r, _):
            pltpu.make_async_copy(table_hbm.at[idx_v.at[r]], rows_v.at[r], sem).wait()
            return 0

        lax.fori_loop(0, R, drain, 0)

        def fix(r, _):
            va = idx_v[r, pl.ds(0, L)]
            vb = idx_v[r, pl.ds(COLS - L, L)]
            ma = jnp.where(va == 0, 0.0, 1.0)
            mb = jnp.where(vb == 0, 0.0, 1.0)
            for j in range(COLS):
                if j < L:
                    b = ma[jnp.full((L,), j, jnp.int32)]
                else:
                    b = mb[jnp.full((L,), j - (COLS - L), jnp.int32)]
                rows_v[r, j, pl.ds(0, L)] = rows_v[r, j, pl.ds(0, L)] * b
                rows_v[r, j, pl.ds(L, L)] = rows_v[r, j, pl.ds(L, L)] * b
            return 0

        lax.fori_loop(0, R, fix, 0)
        pltpu.sync_copy(rows_v, out_hbm.at[pl.ds(cb, R)])
        return 0

    lax.fori_loop(0, N_CHUNKS, chunk_step, 0)


@jax.jit
def _gather(table, x):
    mesh = plsc.VectorSubcoreMesh(core_axis_name="c", subcore_axis_name="s")
    return pl.kernel(
        _body,
        out_type=jax.ShapeDtypeStruct((ROWS, COLS, EMBED), jnp.float32),
        mesh=mesh,
        scratch_types=[
            pltpu.VMEM((R, COLS), jnp.int32),
            pltpu.VMEM((R, COLS, EMBED), jnp.float32),
            pltpu.SemaphoreType.DMA,
        ],
        compiler_params=pltpu.CompilerParams(use_tc_tiling_on_sc=False),
    )(table, x)


def kernel(x, table):
    return _gather(table, x.astype(jnp.int32))


# speedup vs baseline: 1.6470x; 1.0165x over previous
import jax
import jax.numpy as jnp
from jax import lax
from jax.experimental import pallas as pl
from jax.experimental.pallas import tpu as pltpu
from jax.experimental.pallas import tpu_sc as plsc

EMBED = 32
ROWS = 16384
COLS = 26

_info = plsc.get_sparse_core_info()
NC, NS, L = _info.num_cores, _info.num_subcores, _info.num_lanes
NW = NC * NS
ROWS_PER_W = ROWS // NW
R = 64
N_CHUNKS = ROWS_PER_W // R


def _body(table_hbm, x_hbm, out_hbm, idx0_v, idx1_v, rows0_v, rows1_v,
          gsem0, gsem1, osem0, osem1):
    wid = lax.axis_index("s") * NC + lax.axis_index("c")
    base = wid * ROWS_PER_W
    idx = (idx0_v, idx1_v)
    rows = (rows0_v, rows1_v)
    gsem = (gsem0, gsem1)
    osem = (osem0, osem1)

    def stage(g, b):
        cb = base + g * R
        pltpu.sync_copy(x_hbm.at[pl.ds(cb, R)], idx[b])

        def fire(r, _):
            pltpu.async_copy(table_hbm.at[idx[b].at[r]], rows[b].at[r], gsem[b])
            return 0

        lax.fori_loop(0, R, fire, 0)

    def finish(g, b):
        cb = base + g * R

        def drain(r, _):
            pltpu.make_async_copy(
                table_hbm.at[idx[b].at[r]], rows[b].at[r], gsem[b]).wait()
            return 0

        lax.fori_loop(0, R, drain, 0)

        def fix(r, _):
            va = idx[b][r, pl.ds(0, L)]
            vb = idx[b][r, pl.ds(COLS - L, L)]
            ma = jnp.where(va == 0, 0.0, 1.0)
            mb = jnp.where(vb == 0, 0.0, 1.0)
            for j in range(COLS):
                if j < L:
                    m = ma[jnp.full((L,), j, jnp.int32)]
                else:
                    m = mb[jnp.full((L,), j - (COLS - L), jnp.int32)]
                rows[b][r, j, pl.ds(0, L)] = rows[b][r, j, pl.ds(0, L)] * m
                rows[b][r, j, pl.ds(L, L)] = rows[b][r, j, pl.ds(L, L)] * m
            return 0

        lax.fori_loop(0, R, fix, 0)
        pltpu.async_copy(rows[b], out_hbm.at[pl.ds(cb, R)], osem[b])

    def wait_out(g, b):
        cb = base + g * R
        pltpu.make_async_copy(rows[b], out_hbm.at[pl.ds(cb, R)], osem[b]).wait()

    stage(0, 0)
    for g in range(N_CHUNKS):
        b = g % 2
        if g + 1 < N_CHUNKS:
            b2 = (g + 1) % 2
            if g >= 1:
                wait_out(g - 1, b2)
            stage(g + 1, b2)
        finish(g, b)
    if N_CHUNKS >= 2:
        wait_out(N_CHUNKS - 2, (N_CHUNKS - 2) % 2)
    wait_out(N_CHUNKS - 1, (N_CHUNKS - 1) % 2)


@jax.jit
def _gather(table, x):
    mesh = plsc.VectorSubcoreMesh(core_axis_name="c", subcore_axis_name="s")
    return pl.kernel(
        _body,
        out_type=jax.ShapeDtypeStruct((ROWS, COLS, EMBED), jnp.float32),
        mesh=mesh,
        scratch_types=[
            pltpu.VMEM((R, COLS), jnp.int32),
            pltpu.VMEM((R, COLS), jnp.int32),
            pltpu.VMEM((R, COLS, EMBED), jnp.float32),
            pltpu.VMEM((R, COLS, EMBED), jnp.float32),
            pltpu.SemaphoreType.DMA,
            pltpu.SemaphoreType.DMA,
            pltpu.SemaphoreType.DMA,
            pltpu.SemaphoreType.DMA,
        ],
        compiler_params=pltpu.CompilerParams(use_tc_tiling_on_sc=False),
    )(table, x)


def kernel(x, table):
    return _gather(table, x.astype(jnp.int32))
